# baseline (speedup 1.0000x reference)
import functools

import jax
import jax.numpy as jnp
from jax import lax
from jax.experimental import pallas as pl
from jax.experimental.pallas import tpu as pltpu
from jax.experimental.pallas import tpu_sc as plsc

_D = 768
_LANES = 16
_NC = 2
_NS = 16
_NW = _NC * _NS
_C = 16
_NBUF = 3


def _emb_body(tok_hbm, posflat_hbm, tidx_hbm, pidx_hbm, out_hbm,
              posvm, tidx_v, pidx_v,
              t0, t1, t2,
              g0, g1, g2,
              s0, s1, s2, *, per_w):
    wid = lax.axis_index("s") * _NC + lax.axis_index("c")
    base = wid * per_w
    nch = per_w // _C
    tbuf = (t0, t1, t2)
    gsem = (g0, g1, g2)
    ssem = (s0, s1, s2)
    lane = lax.iota(jnp.int32, _LANES)
    zeros = jnp.zeros((_LANES,), jnp.int32)

    pltpu.sync_copy(posflat_hbm, posvm)
    pltpu.sync_copy(tidx_hbm.at[pl.ds(base, per_w)], tidx_v)
    pltpu.sync_copy(pidx_hbm.at[pl.ds(base, per_w)], pidx_v)

    def issue_gather(ci, s):
        pltpu.async_copy(tok_hbm.at[tidx_v.at[pl.ds(ci * _C, _C)]],
                         tbuf[s], gsem[s])

    def wait_gather(ci, s):
        pltpu.make_async_copy(tok_hbm.at[tidx_v.at[pl.ds(ci * _C, _C)]],
                              tbuf[s], gsem[s]).wait()

    def issue_scatter(ci, s):
        pltpu.async_copy(tbuf[s], out_hbm.at[pl.ds(base + ci * _C, _C)],
                         ssem[s])

    def wait_scatter(ci, s):
        pltpu.make_async_copy(tbuf[s],
                              out_hbm.at[pl.ds(base + ci * _C, _C)],
                              ssem[s]).wait()

    def add_chunk(ci, s):
        t = tbuf[s]

        @plsc.parallel_loop(0, _C, step=1, unroll=2)
        def _(i):
            pbro = plsc.load_gather(pidx_v, [zeros + (ci * _C + i)])
            pb = pbro * _D + lane
            for j in range(_D // _LANES):
                pv = plsc.load_gather(posvm, [pb + (j * _LANES)])
                plsc.addupdate(t.at[i, pl.ds(j * _LANES, _LANES)], pv)

    issue_gather(0, 0)
    issue_gather(1, 1)
    issue_gather(2, 2)
    wait_gather(0, 0)
    add_chunk(0, 0)
    issue_scatter(0, 0)
    wait_scatter(0, 0)
    issue_gather(3, 0)
    wait_gather(1, 1)
    add_chunk(1, 1)
    issue_scatter(1, 1)

    def outer(g, _):
        for sp in range(_NBUF):
            ci = 2 + g * _NBUF + sp
            s = (2 + sp) % _NBUF
            s2 = (sp + 1) % _NBUF
            wait_scatter(ci - 1, s2)
            issue_gather(ci + 2, s2)
            wait_gather(ci, s)
            add_chunk(ci, s)
            issue_scatter(ci, s)
        return ()

    lax.fori_loop(0, (nch - 4) // _NBUF, outer, (), unroll=False)

    for ci in (nch - 2, nch - 1):
        s = ci % _NBUF
        wait_scatter(ci - 1, (ci + 2) % _NBUF)
        wait_gather(ci, s)
        add_chunk(ci, s)
        issue_scatter(ci, s)
    wait_scatter(nch - 1, (nch - 1) % _NBUF)


def kernel(token_table, pos_table, tokens, positions):
    b, l = tokens.shape
    bt = b * l
    per_w = bt // _NW
    assert per_w % _C == 0 and (per_w // _C - 4) % _NBUF == 0

    tidx = tokens.reshape(bt).astype(jnp.int32)
    pidx = positions.reshape(bt).astype(jnp.int32)
    posflat = pos_table.reshape(-1)

    mesh = plsc.VectorSubcoreMesh(core_axis_name="c", subcore_axis_name="s")
    body = functools.partial(_emb_body, per_w=per_w)
    run = pl.kernel(
        body,
        mesh=mesh,
        compiler_params=pltpu.CompilerParams(needs_layout_passes=False),
        out_type=jax.ShapeDtypeStruct((bt, _D), jnp.float32),
        scratch_types=[
            pltpu.VMEM((pos_table.size,), jnp.float32),
            pltpu.VMEM((per_w,), jnp.int32),
            pltpu.VMEM((per_w,), jnp.int32),
        ] + [pltpu.VMEM((_C, _D), jnp.float32)] * _NBUF
          + [pltpu.SemaphoreType.DMA] * (2 * _NBUF),
    )
    out = run(token_table, posflat, tidx, pidx)
    return out.reshape(b, l, _D)
